# baseline (speedup 1.0000x reference)
import functools

import jax
import jax.numpy as jnp
from jax import lax
from jax.experimental import pallas as pl
from jax.experimental.pallas import tpu as pltpu
from jax.experimental.pallas import tpu_sc as plsc

VOCAB = 1000
D = 64
BATCH = 4096
SEQ = 200
N = BATCH * SEQ

NC = 2
NS = 16
NW = NC * NS
PER_W = N // NW

IVEC = 128
K = 5
CHUNK = K * IVEC
NCHUNK = PER_W // CHUNK
G = NCHUNK // 2

_mesh = plsc.VectorSubcoreMesh(core_axis_name="c", subcore_axis_name="s")


@functools.partial(
    pl.kernel,
    out_type=jax.ShapeDtypeStruct((N, D), jnp.float32),
    mesh=_mesh,
    scratch_types=[
        pltpu.VMEM((2, CHUNK), jnp.int32),
        pltpu.VMEM((2, CHUNK, D), jnp.float32),
        pltpu.SemaphoreType.DMA,
        pltpu.SemaphoreType.DMA,
        pltpu.SemaphoreType.DMA,
        pltpu.SemaphoreType.DMA,
    ],
    compiler_params=pltpu.CompilerParams(use_tc_tiling_on_sc=False),
)
def _embed_sc(table_hbm, idx_hbm, out_hbm, idx_v, rows_v, g0, g1, o0, o1):
    wid = lax.axis_index("s") * NC + lax.axis_index("c")
    base = wid * PER_W
    gsem = (g0, g1)
    osem = (o0, o1)

    def stage(i, b):
        pltpu.sync_copy(idx_hbm.at[pl.ds(base + i * CHUNK, CHUNK)], idx_v.at[b])
        for j in range(K):
            pltpu.async_copy(
                table_hbm.at[idx_v.at[b].at[pl.ds(j * IVEC, IVEC)]],
                rows_v.at[b].at[pl.ds(j * IVEC, IVEC)],
                gsem[b],
            )

    def drain(sem, b):
        pltpu.make_async_copy(out_hbm.at[pl.ds(0, CHUNK)], rows_v.at[b], sem).wait()

    def flush(i, b):
        drain(gsem[b], b)
        pltpu.async_copy(rows_v.at[b], out_hbm.at[pl.ds(base + i * CHUNK, CHUNK)], osem[b])

    stage(0, 0)

    def body(g, _):
        iA = 2 * g

        @pl.when(g >= 1)
        def _():
            drain(osem[1], 1)
        stage(iA + 1, 1)
        flush(iA, 0)

        @pl.when(g < G - 1)
        def _():
            drain(osem[0], 0)
            stage(iA + 2, 0)
        flush(iA + 1, 1)
        return ()

    lax.fori_loop(0, G, body, ())

    drain(osem[0], 0)
    drain(osem[1], 1)


def kernel(table, inp):
    idx = inp.reshape(N)
    out = _embed_sc(table, idx)
    return out.reshape(BATCH, SEQ, D)


# speedup vs baseline: 1.4244x; 1.4244x over previous
import functools

import jax
import jax.numpy as jnp
from jax import lax
from jax.experimental import pallas as pl
from jax.experimental.pallas import tpu as pltpu
from jax.experimental.pallas import tpu_sc as plsc

VOCAB = 1000
D = 64
BATCH = 4096
SEQ = 200
N = BATCH * SEQ

NC = 2
NS = 16
NW = NC * NS
PER_W = N // NW

CHUNK = 160
SBLK = 5120
NSTAGE = PER_W // SBLK
CPS = SBLK // CHUNK
UNROLL = 16
L = 16

_mesh = plsc.VectorSubcoreMesh(core_axis_name="c", subcore_axis_name="s")


@functools.partial(
    pl.kernel,
    out_type=jax.ShapeDtypeStruct((N, D), jnp.float32),
    mesh=_mesh,
    scratch_types=[
        pltpu.VMEM((VOCAB * D,), jnp.float32),
        pltpu.VMEM((SBLK,), jnp.int32),
        pltpu.VMEM((2, CHUNK, D), jnp.float32),
        pltpu.SemaphoreType.DMA,
        pltpu.SemaphoreType.DMA,
    ],
    compiler_params=pltpu.CompilerParams(internal_scratch_in_bytes=4096),
)
def _embed_sc(table_hbm, idx_hbm, out_hbm, table_v, idx_v, rows_v, o0, o1):
    wid = lax.axis_index("s") * NC + lax.axis_index("c")
    base = wid * PER_W
    osem = (o0, o1)

    pltpu.sync_copy(table_hbm, table_v)

    def drain(sem, b):
        pltpu.make_async_copy(out_hbm.at[pl.ds(0, CHUNK)], rows_v.at[b], sem).wait()

    def assemble(lc, b):
        cbase = lc * CHUNK

        def group(g2, _):
            rb = g2 * UNROLL
            iv = idx_v[pl.ds(cbase + rb, UNROLL)] * D
            for r in range(UNROLL):
                off = iv[r]
                for c in range(D // L):
                    rows_v[b, rb + r, pl.ds(c * L, L)] = table_v[pl.ds(off + c * L, L)]
            return ()

        lax.fori_loop(0, CHUNK // UNROLL, group, ())

    def stage(s, _):
        pltpu.sync_copy(idx_hbm.at[pl.ds(base + s * SBLK, SBLK)], idx_v)

        def body(g, _):
            for b in range(2):
                lc = 2 * g + b
                ci = s * CPS + lc

                @pl.when(ci >= 2)
                def _():
                    drain(osem[b], b)

                assemble(lc, b)
                pltpu.async_copy(
                    rows_v.at[b], out_hbm.at[pl.ds(base + ci * CHUNK, CHUNK)], osem[b]
                )
            return ()

        lax.fori_loop(0, CPS // 2, body, ())
        return ()

    lax.fori_loop(0, NSTAGE, stage, ())

    for b in range(2):
        drain(osem[b], b)


def kernel(table, inp):
    idx = inp.reshape(N)
    out = _embed_sc(table.reshape(VOCAB * D), idx)
    return out.reshape(BATCH, SEQ, D)
